# speedup vs baseline: 1.4698x; 1.4698x over previous
import functools

import jax
import jax.numpy as jnp
from jax import lax
from jax.experimental import pallas as pl
from jax.experimental.pallas import tpu as pltpu
from jax.experimental.pallas import tpu_sc as plsc

_H = 512
_B = 16384
_K = 64
_NW = 32
_SPW = _B // _NW
_HC = _H // 16


def _sc_diff(idx_all, ft_weight):
    mesh = plsc.VectorSubcoreMesh(core_axis_name="c", subcore_axis_name="s")

    @functools.partial(
        pl.kernel,
        out_type=jax.ShapeDtypeStruct((_B, _H), jnp.float32),
        mesh=mesh,
        scratch_types=[
            pltpu.VMEM((2 * _K,), jnp.int32),
            pltpu.VMEM((2 * _K, _H), jnp.float32),
            pltpu.VMEM((_H,), jnp.float32),
            pltpu.SemaphoreType.DMA,
        ],
    )
    def k(idx_hbm, table_hbm, out_hbm, idx_v, buf_v, acc_v, sem):
        wid = lax.axis_index("s") * 2 + lax.axis_index("c")
        base = wid * _SPW

        def sample_body(s, carry):
            gs = base + s
            pltpu.sync_copy(idx_hbm.at[gs], idx_v)
            pltpu.async_copy(table_hbm.at[idx_v], buf_v, sem).wait()

            def row_add(r, acc):
                return tuple(acc[h] + buf_v[r, pl.ds(16 * h, 16)]
                             for h in range(_HC))

            def row_sub(r, acc):
                return tuple(acc[h] - buf_v[r, pl.ds(16 * h, 16)]
                             for h in range(_HC))

            acc0 = tuple(jnp.zeros((16,), jnp.float32) for _ in range(_HC))
            acc = lax.fori_loop(0, _K, row_add, acc0)
            acc = lax.fori_loop(_K, 2 * _K, row_sub, acc)
            for h in range(_HC):
                acc_v[pl.ds(16 * h, 16)] = acc[h]
            pltpu.sync_copy(acc_v, out_hbm.at[gs])
            return carry

        lax.fori_loop(0, _SPW, sample_body, 0)

    return k(idx_all, ft_weight)


def _tc_head(diff, mult, w1t, b1, w2t, b2, wo_row, bo):
    blk = 2048

    def body(diff_ref, mult_ref, w1_ref, b1_ref, w2_ref, b2_ref, wo_ref,
             bo_ref, out_ref):
        x = jnp.maximum(diff_ref[...] * mult_ref[...], 0.0)
        h1 = jnp.dot(x, w1_ref[...], preferred_element_type=jnp.float32)
        h1 = jnp.maximum((h1 + b1_ref[...]) * (1.0 / 64.0), 0.0)
        h2 = jnp.dot(h1, w2_ref[...], preferred_element_type=jnp.float32)
        h2 = jnp.maximum((h2 + b2_ref[...]) * (1.0 / 64.0), 0.0)
        o = jnp.sum(h2 * wo_ref[...], axis=1, keepdims=True)
        out_ref[...] = (o + bo_ref[...]) * (1.0 / 16.0)

    grid = (_B // blk,)
    full = lambda shape: pl.BlockSpec(shape, lambda i: (0, 0))
    return pl.pallas_call(
        body,
        grid=grid,
        in_specs=[
            pl.BlockSpec((blk, _H), lambda i: (i, 0)),
            pl.BlockSpec((blk, 1), lambda i: (i, 0)),
            full((_H, 32)),
            full((1, 32)),
            full((32, 32)),
            full((1, 32)),
            full((1, 32)),
            full((1, 1)),
        ],
        out_specs=pl.BlockSpec((blk, 1), lambda i: (i, 0)),
        out_shape=jax.ShapeDtypeStruct((_B, 1), jnp.float32),
    )(diff, mult, w1t, b1, w2t, b2, wo_row, bo)


def kernel(white_idx, black_idx, stm, ft_weight, ft_bias, w1, b1, w2, b2,
           wo, bo):
    idx_all = jnp.concatenate(
        [white_idx.astype(jnp.int32), black_idx.astype(jnp.int32)], axis=1)
    diff = _sc_diff(idx_all, ft_weight)
    sign = jnp.where(stm == 0, 1.0, -1.0).astype(jnp.float32)
    mult = (sign * (1.0 / 64.0))[:, None]
    out = _tc_head(diff, mult, w1.T, b1[None, :], w2.T, b2[None, :],
                   wo, bo[None, :])
    return out[:, 0]
